# baseline (speedup 1.0000x reference)
import functools
import math

import jax
import jax.numpy as jnp
from jax.experimental import pallas as pl
from jax.experimental.pallas import tpu as pltpu


def _body(shift, mask, som_dim, idx_hbm, out_hbm, sd_ref, kx_hbm, ky_hbm,
          vin, vkx, vky, sem_pass, sem_in, sem_kx, sem_ky):
    cp_pass = pltpu.make_async_copy(idx_hbm, out_hbm, sem_pass)
    cp_pass.start()
    cp_in = pltpu.make_async_copy(idx_hbm, vin, sem_in)
    cp_in.start()
    sd_ref[()] = jnp.float32(som_dim)
    cp_in.wait()
    v = vin[...]
    vkx[...] = (v >> shift).astype(jnp.float32)
    vky[...] = (v & mask).astype(jnp.float32)
    cp_kx = pltpu.make_async_copy(vkx, kx_hbm, sem_kx)
    cp_ky = pltpu.make_async_copy(vky, ky_hbm, sem_ky)
    cp_kx.start()
    cp_ky.start()
    cp_kx.wait()
    cp_ky.wait()
    cp_pass.wait()


def kernel(all_codebook_idxs, distance_matrix):
    som_dim = math.sqrt(distance_matrix.shape[-1])
    som_dim_i = int(round(som_dim))
    shift = som_dim_i.bit_length() - 1
    assert (1 << shift) == som_dim_i
    body = functools.partial(_body, shift, som_dim_i - 1, som_dim)
    shape = all_codebook_idxs.shape
    f32 = jnp.float32
    idx32 = all_codebook_idxs.astype(jnp.int32)
    hbm = pl.BlockSpec(memory_space=pl.ANY)
    out, sd, kx, ky = pl.pallas_call(
        body,
        in_specs=[hbm],
        out_shape=(jax.ShapeDtypeStruct(shape, idx32.dtype),
                   jax.ShapeDtypeStruct((), f32),
                   jax.ShapeDtypeStruct(shape, f32),
                   jax.ShapeDtypeStruct(shape, f32)),
        out_specs=(hbm,
                   pl.BlockSpec(memory_space=pltpu.SMEM),
                   hbm,
                   hbm),
        scratch_shapes=[
            pltpu.VMEM(shape, jnp.int32),
            pltpu.VMEM(shape, f32),
            pltpu.VMEM(shape, f32),
            pltpu.SemaphoreType.DMA,
            pltpu.SemaphoreType.DMA,
            pltpu.SemaphoreType.DMA,
            pltpu.SemaphoreType.DMA,
        ],
    )(idx32)
    return (out.astype(all_codebook_idxs.dtype), sd, kx, ky)


# speedup vs baseline: 1.9301x; 1.9301x over previous
import functools
import math

import jax
import jax.numpy as jnp
from jax.experimental import pallas as pl
from jax.experimental.pallas import tpu as pltpu


def _body_pow2(shift, mask, som_dim, idx_ref, out_ref, sd_ref, kx_ref, ky_ref):
    v = idx_ref[...]
    out_ref[...] = v
    sd_ref[()] = jnp.float32(som_dim)
    kx_ref[...] = (v >> shift).astype(jnp.float32)
    ky_ref[...] = (v & mask).astype(jnp.float32)


def _body_general(som_dim_i, som_dim, idx_ref, out_ref, sd_ref, kx_ref, ky_ref):
    v = idx_ref[...]
    out_ref[...] = v
    sd_ref[()] = jnp.float32(som_dim)
    q = v // som_dim_i
    kx_ref[...] = q.astype(jnp.float32)
    ky_ref[...] = (v - q * som_dim_i).astype(jnp.float32)


def kernel(all_codebook_idxs, distance_matrix):
    som_dim = math.sqrt(distance_matrix.shape[-1])
    som_dim_i = int(round(som_dim))
    shift = som_dim_i.bit_length() - 1
    if (1 << shift) == som_dim_i:
        body = functools.partial(_body_pow2, shift, som_dim_i - 1, som_dim)
    else:
        body = functools.partial(_body_general, som_dim_i, som_dim)
    shape = all_codebook_idxs.shape
    f32 = jnp.float32
    idx32 = all_codebook_idxs.astype(jnp.int32)
    out, sd, kx, ky = pl.pallas_call(
        body,
        in_specs=[pl.BlockSpec(shape, lambda: (0, 0))],
        out_shape=(jax.ShapeDtypeStruct(shape, idx32.dtype),
                   jax.ShapeDtypeStruct((), f32),
                   jax.ShapeDtypeStruct(shape, f32),
                   jax.ShapeDtypeStruct(shape, f32)),
        out_specs=(pl.BlockSpec(shape, lambda: (0, 0)),
                   pl.BlockSpec(memory_space=pltpu.SMEM),
                   pl.BlockSpec(shape, lambda: (0, 0)),
                   pl.BlockSpec(shape, lambda: (0, 0))),
    )(idx32)
    return (out.astype(all_codebook_idxs.dtype),
            sd, kx, ky)
